# Initial kernel scaffold; baseline (speedup 1.0000x reference)
import jax
import jax.numpy as jnp
from jax.experimental import pallas as pl


def kernel(token_ids, idf_weights):
    raise NotImplementedError("write your pallas kernel here")


# speedup vs baseline: 23.9215x; 23.9215x over previous
import functools

import jax
import jax.numpy as jnp
from jax import lax
from jax.experimental import pallas as pl
from jax.experimental.pallas import tpu as pltpu
from jax.experimental.pallas import tpu_sc as plsc

B, L, V = 4096, 200, 1000
VP = 1024
LANES = 16
NC, NS = 2, 16
NW = NC * NS
RPW = B // NW
GROUPS = RPW // LANES


def _tfidf_body(tok_hbm, idf_hbm, out_hbm, tok_v, idf_v, hist):
    wid = lax.axis_index("s") * NC + lax.axis_index("c")
    base = wid * RPW

    pltpu.sync_copy(tok_hbm.at[pl.ds(base, RPW), :], tok_v)
    pltpu.sync_copy(idf_hbm, idf_v)

    lanes = lax.iota(jnp.int32, 16)
    zf = jnp.zeros((16,), jnp.float32)

    for g in range(GROUPS):
        def zbody(i, _):
            off = i * 16
            for r in range(LANES):
                hist[r, pl.ds(off, 16)] = zf
            return _

        lax.fori_loop(0, VP // 16, zbody, None)

        rows = lanes + g * LANES

        def tbody(l, _):
            lv = jnp.full((16,), l, dtype=jnp.int32)
            tok = plsc.load_gather(tok_v, [rows, lv])
            w = plsc.load_gather(idf_v, [tok])
            plsc.addupdate_scatter(hist, [lanes, tok], w)
            return _

        lax.fori_loop(0, L, tbody, None)

        pltpu.sync_copy(
            hist.at[:, pl.ds(0, V)],
            out_hbm.at[pl.ds(base + g * LANES, LANES), :],
        )


_tfidf = functools.partial(
    pl.kernel,
    out_type=jax.ShapeDtypeStruct((B, V), jnp.float32),
    mesh=plsc.VectorSubcoreMesh(core_axis_name="c", subcore_axis_name="s"),
    compiler_params=pltpu.CompilerParams(
        use_tc_tiling_on_sc=False, needs_layout_passes=False
    ),
    scratch_types=[
        pltpu.VMEM((RPW, L), jnp.int32),
        pltpu.VMEM((V,), jnp.float32),
        pltpu.VMEM((LANES, VP), jnp.float32),
    ],
)(_tfidf_body)


def kernel(token_ids, idf_weights):
    return _tfidf(token_ids, idf_weights)


# speedup vs baseline: 24.9115x; 1.0414x over previous
import functools

import jax
import jax.numpy as jnp
from jax import lax
from jax.experimental import pallas as pl
from jax.experimental.pallas import tpu as pltpu
from jax.experimental.pallas import tpu_sc as plsc

B, L, V = 4096, 200, 1000
VP = 1024
LANES = 16
NC, NS = 2, 16
NW = NC * NS
RPW = B // NW
GROUPS = RPW // LANES
UNROLL = 8


def _tfidf_body(tok_hbm, idf_hbm, out_hbm, tok_v, idf_v, hist, sem0, sem1):
    wid = lax.axis_index("s") * NC + lax.axis_index("c")
    base = wid * RPW

    pltpu.sync_copy(tok_hbm.at[pl.ds(base, RPW), :], tok_v)
    pltpu.sync_copy(idf_hbm, idf_v)

    lanes = lax.iota(jnp.int32, 16)
    zf = jnp.zeros((16,), jnp.float32)
    sems = (sem0, sem1)

    def out_copy(g):
        return pltpu.make_async_copy(
            hist.at[g % 2, :, pl.ds(0, V)],
            out_hbm.at[pl.ds(base + g * LANES, LANES), :],
            sems[g % 2],
        )

    for g in range(GROUPS):
        h = hist.at[g % 2]
        if g >= 2:
            out_copy(g - 2).wait()

        def zbody(i, _):
            off = i * 32
            for r in range(LANES):
                h[r, pl.ds(off, 16)] = zf
                h[r, pl.ds(off + 16, 16)] = zf
            return _

        lax.fori_loop(0, VP // 32, zbody, None)

        rows = lanes + g * LANES

        def tbody(i, _):
            l0 = i * UNROLL
            for j in range(UNROLL):
                lv = jnp.full((16,), l0 + j, dtype=jnp.int32)
                tok = plsc.load_gather(tok_v, [rows, lv])
                w = plsc.load_gather(idf_v, [tok])
                plsc.addupdate_scatter(h, [lanes, tok], w)
            return _

        lax.fori_loop(0, L // UNROLL, tbody, None)

        out_copy(g).start()

    out_copy(GROUPS - 2).wait()
    out_copy(GROUPS - 1).wait()


_tfidf = functools.partial(
    pl.kernel,
    out_type=jax.ShapeDtypeStruct((B, V), jnp.float32),
    mesh=plsc.VectorSubcoreMesh(core_axis_name="c", subcore_axis_name="s"),
    compiler_params=pltpu.CompilerParams(
        use_tc_tiling_on_sc=False, needs_layout_passes=False
    ),
    scratch_types=[
        pltpu.VMEM((RPW, L), jnp.int32),
        pltpu.VMEM((V,), jnp.float32),
        pltpu.VMEM((2, LANES, VP), jnp.float32),
        pltpu.SemaphoreType.DMA,
        pltpu.SemaphoreType.DMA,
    ],
)(_tfidf_body)


def kernel(token_ids, idf_weights):
    return _tfidf(token_ids, idf_weights)
